# baseline (speedup 1.0000x reference)
import functools

import jax
import jax.numpy as jnp
from jax import lax
from jax.experimental import pallas as pl
from jax.experimental.pallas import tpu as pltpu
from jax.experimental.pallas import tpu_sc as plsc

B, S, D = 4096, 200, 64
L = 16
NC, NS = 2, 16
NW = NC * NS
ROWS = B * S
RPW = ROWS // NW
SEQ_PER_CHUNK = 4
CHUNK = SEQ_PER_CHUNK * S
NCHUNK = RPW // CHUNK
NSUB = 8
SUB = CHUNK // NSUB

_mesh = plsc.VectorSubcoreMesh(
    core_axis_name="c", subcore_axis_name="s", num_cores=NC, num_subcores=NS
)


@functools.partial(
    pl.kernel,
    out_type=jax.ShapeDtypeStruct((ROWS, D), jnp.float32),
    mesh=_mesh,
    scratch_types=[
        pltpu.VMEM((NSUB, SUB), jnp.int32),
        pltpu.VMEM((CHUNK, D), jnp.float32),
        pltpu.VMEM((S, D), jnp.float32),
        pltpu.SemaphoreType.DMA,
    ],
    compiler_params=pltpu.CompilerParams(use_tc_tiling_on_sc=False),
)
def _embed(x_hbm, tok_hbm, pos_hbm, out_hbm, idx_v, rows_v, pos_v, sem):
    wid = lax.axis_index("s") * NC + lax.axis_index("c")
    pltpu.sync_copy(pos_hbm, pos_v)

    def chunk_body(c, carry):
        base = wid * RPW + c * CHUNK
        pltpu.sync_copy(x_hbm.at[wid, c], idx_v)
        descs = [
            pltpu.async_copy(
                tok_hbm.at[idx_v.at[j]], rows_v.at[pl.ds(j * SUB, SUB)], sem
            )
            for j in range(NSUB)
        ]
        for d in descs:
            d.wait()

        def pos_body(p, carry2):
            for j in range(D // L):
                pv = pos_v[p, pl.ds(j * L, L)]
                for s0 in range(SEQ_PER_CHUNK):
                    rows_v[s0 * S + p, pl.ds(j * L, L)] += pv
            return carry2

        lax.fori_loop(0, S, pos_body, 0)
        pltpu.sync_copy(rows_v, out_hbm.at[pl.ds(base, CHUNK)])
        return carry

    lax.fori_loop(0, NCHUNK, chunk_body, 0)


def kernel(x, token_table, pos_table):
    x4 = x.reshape(NW, NCHUNK, NSUB, SUB)
    out = _embed(x4, token_table, pos_table[:S])
    return out.reshape(B, S, D)


# speedup vs baseline: 1.0031x; 1.0031x over previous
import functools

import jax
import jax.numpy as jnp
from jax import lax
from jax.experimental import pallas as pl
from jax.experimental.pallas import tpu as pltpu
from jax.experimental.pallas import tpu_sc as plsc

B, S, D = 4096, 200, 64
L = 16
NC, NS = 2, 16
NW = NC * NS
SEQ_PER_W = B // NW
SEQ_PER_CHUNK = 4
CHUNK = SEQ_PER_CHUNK * S
NCHUNK = SEQ_PER_W // SEQ_PER_CHUNK
SPLITS = ((0, 120), (120, 80))

_mesh = plsc.VectorSubcoreMesh(
    core_axis_name="c", subcore_axis_name="s", num_cores=NC, num_subcores=NS
)


@functools.partial(
    pl.kernel,
    out_type=jax.ShapeDtypeStruct((B, S, D), jnp.float32),
    mesh=_mesh,
    scratch_types=[
        pltpu.VMEM((SEQ_PER_CHUNK, S), jnp.int32),
        pltpu.VMEM((SEQ_PER_CHUNK, S, D), jnp.float32),
        pltpu.VMEM((S, D), jnp.float32),
        pltpu.SemaphoreType.DMA,
    ],
    compiler_params=pltpu.CompilerParams(use_tc_tiling_on_sc=False),
)
def _embed(x_hbm, tok_hbm, pos_hbm, out_hbm, idx_v, rows_v, pos_v, sem):
    wid = lax.axis_index("s") * NC + lax.axis_index("c")
    pltpu.sync_copy(pos_hbm, pos_v)

    def chunk_body(c, carry):
        seq0 = wid * SEQ_PER_W + c * SEQ_PER_CHUNK
        pltpu.sync_copy(x_hbm.at[pl.ds(seq0, SEQ_PER_CHUNK)], idx_v)
        descs = [
            pltpu.async_copy(
                tok_hbm.at[idx_v.at[s0, pl.ds(off, n)]],
                rows_v.at[s0, pl.ds(off, n)],
                sem,
            )
            for s0 in range(SEQ_PER_CHUNK)
            for off, n in SPLITS
        ]
        for d in descs:
            d.wait()

        def pos_body(p, carry2):
            for j in range(D // L):
                pv = pos_v[p, pl.ds(j * L, L)]
                for s0 in range(SEQ_PER_CHUNK):
                    rows_v[s0, p, pl.ds(j * L, L)] += pv
            return carry2

        lax.fori_loop(0, S, pos_body, 0)
        pltpu.sync_copy(rows_v, out_hbm.at[pl.ds(seq0, SEQ_PER_CHUNK)])
        return carry

    lax.fori_loop(0, NCHUNK, chunk_body, 0)


def kernel(x, token_table, pos_table):
    return _embed(x, token_table, pos_table[:S])
